# baseline (speedup 1.0000x reference)
import functools

import jax
import jax.numpy as jnp
from jax import lax
from jax.experimental import pallas as pl
from jax.experimental.pallas import tpu as pltpu
from jax.experimental.pallas import tpu_sc as plsc

_NC = 2
_NS = 16
_NW = _NC * _NS
_L = 16
_CH = 128

_LN2 = 0.6931471805599453
_SQRT2 = 1.4142135381698608


def _ln(y):
    yi = lax.bitcast_convert_type(y, jnp.int32)
    ex = lax.shift_right_arithmetic(yi, 23) - 127
    mi = lax.bitwise_or(lax.bitwise_and(yi, 0x007FFFFF), 0x3F800000)
    m = lax.bitcast_convert_type(mi, jnp.float32)
    big = m >= _SQRT2
    m = jnp.where(big, m * jnp.float32(0.5), m)
    e = ex.astype(jnp.float32) + jnp.where(big, jnp.float32(1.0), jnp.float32(0.0))
    f = m - jnp.float32(1.0)
    q = jnp.float32(1.0 / 9.0)
    for c in (-1.0 / 8, 1.0 / 7, -1.0 / 6, 1.0 / 5, -1.0 / 4, 1.0 / 3,
              -1.0 / 2, 1.0):
        q = q * f + jnp.float32(c)
    return e * jnp.float32(_LN2) + f * q


def _sc_body(nch, d, hwl_hbm, table_hbm, out_hbm, hwl_v, idx_v, rows_v, out_v, sem):
    cid = lax.axis_index("c")
    sid = lax.axis_index("s")
    wid = sid * _NC + cid
    kc = d // _L
    bpw = nch * _CH

    pltpu.sync_copy(hwl_hbm.at[wid], hwl_v)
    lanes3 = lax.iota(jnp.int32, _L) * 3
    for c in range(nch):
        for g in range(_CH // _L):
            base = (c * _CH + g * _L) * 3
            for j in range(3):
                idx_v[pl.ds((c * 3 + j) * _CH + g * _L, _L)] = plsc.load_gather(
                    hwl_v, [lanes3 + (base + j)]
                )

    def _idx_ref(c, j):
        return idx_v.at[pl.ds((c * 3 + j) * _CH, _CH)]

    def fetch(c, buf):
        for j in range(3):
            pltpu.async_copy(
                table_hbm.at[_idx_ref(c, j)], rows_v.at[buf, j], sem.at[buf]
            )

    def drain(c, buf):
        for j in range(3):
            pltpu.make_async_copy(
                table_hbm.at[_idx_ref(c, j)], rows_v.at[buf, j], sem.at[buf]
            ).wait()

    def compute_chunk(c, buf):
        drain(c, buf)
        lanes = lax.iota(jnp.int32, _L)

        def group_body(g, carry2):
            dv = jnp.zeros((_L,), jnp.float32)
            for tt in range(_L):
                t = g * _L + tt
                acc = jnp.zeros((_L,), jnp.float32)
                for k in range(kc):
                    hv = rows_v[buf, 0, t, pl.ds(k * _L, _L)]
                    wv = rows_v[buf, 1, t, pl.ds(k * _L, _L)]
                    lv = rows_v[buf, 2, t, pl.ds(k * _L, _L)]
                    hw = hv - wv
                    hl = hv - lv
                    acc = acc + (hw + hl) * (hw - hl)
                dv = jnp.where(lanes == tt, jnp.sum(acc), dv)
            y = jnp.float32(1.0) + jnp.exp(dv)
            out_v[pl.ds(c * _CH + g * _L, _L)] = _ln(y)
            return carry2

        lax.fori_loop(0, _CH // _L, group_body, 0)

    fetch(0, 0)

    def pair_body(p, carry):
        c0 = p * 2

        @pl.when(c0 + 1 < nch)
        def _():
            fetch(c0 + 1, 1)

        compute_chunk(c0, 0)

        @pl.when(c0 + 2 < nch)
        def _():
            fetch(c0 + 2, 0)

        @pl.when(c0 + 1 < nch)
        def _():
            compute_chunk(c0 + 1, 1)

        return carry

    lax.fori_loop(0, (nch + 1) // 2, pair_body, 0)
    pltpu.sync_copy(out_v, out_hbm.at[pl.ds(wid * (nch * _CH), nch * _CH)])


def kernel(h_w_l, embedding):
    b = h_w_l.shape[0]
    n, d = embedding.shape
    bpw = b // _NW
    nch = bpw // _CH

    mesh = plsc.VectorSubcoreMesh(core_axis_name="c", subcore_axis_name="s")
    fn = pl.kernel(
        functools.partial(_sc_body, nch, d),
        out_type=jax.ShapeDtypeStruct((b,), jnp.float32),
        mesh=mesh,
        compiler_params=pltpu.CompilerParams(needs_layout_passes=False),
        scratch_types=[
            pltpu.VMEM((bpw * 3,), jnp.int32),
            pltpu.VMEM((nch * 3 * _CH,), jnp.int32),
            pltpu.VMEM((2, 3, _CH, d), jnp.float32),
            pltpu.VMEM((bpw,), jnp.float32),
            pltpu.SemaphoreType.DMA((2,)),
        ],
    )
    return fn(h_w_l.reshape(_NW, bpw * 3), embedding)


# speedup vs baseline: 1.2659x; 1.2659x over previous
import functools

import jax
import jax.numpy as jnp
from jax import lax
from jax.experimental import pallas as pl
from jax.experimental.pallas import tpu as pltpu
from jax.experimental.pallas import tpu_sc as plsc

_NC = 2
_NS = 16
_NW = _NC * _NS
_L = 16
_CH = 128

_LN2 = 0.6931471805599453
_SQRT2 = 1.4142135381698608


def _ln(y):
    yi = lax.bitcast_convert_type(y, jnp.int32)
    ex = lax.shift_right_arithmetic(yi, 23) - 127
    mi = lax.bitwise_or(lax.bitwise_and(yi, 0x007FFFFF), 0x3F800000)
    m = lax.bitcast_convert_type(mi, jnp.float32)
    big = m >= _SQRT2
    m = jnp.where(big, m * jnp.float32(0.5), m)
    e = ex.astype(jnp.float32) + jnp.where(big, jnp.float32(1.0), jnp.float32(0.0))
    f = m - jnp.float32(1.0)
    q = jnp.float32(1.0 / 9.0)
    for c in (-1.0 / 8, 1.0 / 7, -1.0 / 6, 1.0 / 5, -1.0 / 4, 1.0 / 3,
              -1.0 / 2, 1.0):
        q = q * f + jnp.float32(c)
    return e * jnp.float32(_LN2) + f * q


def _sc_body(nch, d, idx_hbm, table_hbm, out_hbm, idx_v, rows_v, out_v, sem):
    cid = lax.axis_index("c")
    sid = lax.axis_index("s")
    wid = sid * _NC + cid
    kc = d // _L

    pltpu.sync_copy(idx_hbm.at[wid], idx_v)

    def fetch(c, buf):
        for j in range(3):
            pltpu.async_copy(
                table_hbm.at[idx_v.at[c, j]], rows_v.at[buf, j], sem.at[buf]
            )

    def drain(c, buf):
        for j in range(3):
            pltpu.make_async_copy(
                table_hbm.at[idx_v.at[c, j]], rows_v.at[buf, j], sem.at[buf]
            ).wait()

    def compute_chunk(c, buf):
        drain(c, buf)
        lanes = lax.iota(jnp.int32, _L)

        def group_body(g, carry2):
            dv = jnp.zeros((_L,), jnp.float32)
            for tt in range(_L):
                t = g * _L + tt
                acc = jnp.zeros((_L,), jnp.float32)
                for k in range(kc):
                    hv = rows_v[buf, 0, t, pl.ds(k * _L, _L)]
                    wv = rows_v[buf, 1, t, pl.ds(k * _L, _L)]
                    lv = rows_v[buf, 2, t, pl.ds(k * _L, _L)]
                    hw = hv - wv
                    hl = hv - lv
                    acc = acc + (hw + hl) * (hw - hl)
                dv = jnp.where(lanes == tt, jnp.sum(acc), dv)
            y = jnp.float32(1.0) + jnp.exp(dv)
            out_v[pl.ds(c * _CH + g * _L, _L)] = _ln(y)
            return carry2

        lax.fori_loop(0, _CH // _L, group_body, 0)

    fetch(0, 0)

    def pair_body(p, carry):
        c0 = p * 2

        @pl.when(c0 + 1 < nch)
        def _():
            fetch(c0 + 1, 1)

        compute_chunk(c0, 0)

        @pl.when(c0 + 2 < nch)
        def _():
            fetch(c0 + 2, 0)

        @pl.when(c0 + 1 < nch)
        def _():
            compute_chunk(c0 + 1, 1)

        return carry

    lax.fori_loop(0, (nch + 1) // 2, pair_body, 0)
    pltpu.sync_copy(out_v, out_hbm.at[pl.ds(wid * (nch * _CH), nch * _CH)])


def kernel(h_w_l, embedding):
    b = h_w_l.shape[0]
    n, d = embedding.shape
    bpw = b // _NW
    nch = bpw // _CH

    mesh = plsc.VectorSubcoreMesh(core_axis_name="c", subcore_axis_name="s")
    fn = pl.kernel(
        functools.partial(_sc_body, nch, d),
        out_type=jax.ShapeDtypeStruct((b,), jnp.float32),
        mesh=mesh,
        compiler_params=pltpu.CompilerParams(needs_layout_passes=False),
        scratch_types=[
            pltpu.VMEM((nch, 3, _CH), jnp.int32),
            pltpu.VMEM((2, 3, _CH, d), jnp.float32),
            pltpu.VMEM((bpw,), jnp.float32),
            pltpu.SemaphoreType.DMA((2,)),
        ],
    )
    idx_all = h_w_l.reshape(_NW, nch, _CH, 3).transpose(0, 1, 3, 2)
    return fn(idx_all, embedding)
